# baseline (speedup 1.0000x reference)
import functools

import jax
import jax.numpy as jnp
from jax import lax
from jax.experimental import pallas as pl
from jax.experimental.pallas import tpu as pltpu
from jax.experimental.pallas import tpu_sc as plsc

N = 10000
E = 320000
D = 128
DE = 16
H = 128

NSLOT = 64
NLOC = 160
NPAD = NSLOT * NLOC
CHUNK = 1280
NCHUNK = E // CHUNK
NV = CHUNK // 16
SUB = 64
CAP = 2048
MASKC = CAP - 1
TRASH = CAP
CBUF = CAP + 16
FBIG = 3.0e38


def _edge_body(src_hbm, dst_hbm, a_hbm, b_hbm, c_hbm,
               ssum_hbm, ssq_hbm, smn_hbm, smx_hbm, deg_hbm,
               dstb, srcb, idc, srcc, dlocc, bufa, bufc, bloc,
               accs, accq, accmn, accmx, accd, sem_a, sem_c):
    wid = lax.axis_index("s") * 2 + lax.axis_index("c")

    zero16 = jnp.zeros((16,), jnp.float32)
    pos16 = jnp.full((16,), FBIG, jnp.float32)
    neg16 = jnp.full((16,), -FBIG, jnp.float32)
    izero16 = jnp.zeros((16,), jnp.int32)

    def _initpad(t, c):
        idc[pl.ds(t * 16, 16)] = izero16
        srcc[pl.ds(t * 16, 16)] = izero16
        dlocc[pl.ds(t * 16, 16)] = izero16
        return c
    lax.fori_loop(0, CBUF // 16, _initpad, 0)

    for p in range(2):
        slot = p * 32 + wid
        lo = slot * NLOC

        def _initacc(t, c):
            o = pl.ds(t * 16, 16)
            accs[o] = zero16
            accq[o] = zero16
            accmn[o] = pos16
            accmx[o] = neg16
            return c
        lax.fori_loop(0, NLOC * H // 16, _initacc, 0)

        def _initd(t, c):
            accd[pl.ds(t * 16, 16)] = zero16
            return c
        lax.fori_loop(0, (NLOC + 16) // 16, _initd, 0)

        pltpu.sync_copy(b_hbm.at[pl.ds(lo * H, NLOC * H)], bloc)

        one16 = jnp.where(lax.iota(jnp.int32, 16) == 0, 1.0, 0.0)

        def _window(r0, n):
            cp_a = pltpu.async_copy(
                a_hbm.at[srcc.at[pl.ds(r0, SUB)]], bufa, sem_a)
            cp_c = pltpu.async_copy(
                c_hbm.at[idc.at[pl.ds(r0, SUB)]], bufc, sem_c)
            cp_a.wait()
            cp_c.wait()

            def _edge(i, c2):
                row = dlocc[pl.ds(r0 + i, 16)][0]
                accd[pl.ds(row, 16)] = accd[pl.ds(row, 16)] + one16
                rb = row * H
                for j in range(H // 16):
                    o = pl.ds(rb + j * 16, 16)
                    a = bufa[i, pl.ds(j * 16, 16)]
                    cc = bufc[i, pl.ds(j * 16, 16)]
                    b = bloc[pl.ds(rb + j * 16, 16)]
                    m = jnp.maximum(a + b + cc, 0.0)
                    accs[o] = accs[o] + m
                    accq[o] = accq[o] + m * m
                    accmn[o] = jnp.minimum(accmn[o], m)
                    accmx[o] = jnp.maximum(accmx[o], m)
                return c2
            lax.fori_loop(0, n, _edge, 0)

        def _chunk(ci, wr):
            w0, r0 = wr
            g = ci * CHUNK
            cp_d = pltpu.async_copy(dst_hbm.at[pl.ds(g, CHUNK)], dstb, sem_a)
            cp_s = pltpu.async_copy(src_hbm.at[pl.ds(g, CHUNK)], srcb, sem_c)
            cp_d.wait()
            cp_s.wait()

            def _filt(v, wv):
                d = dstb[pl.ds(v * 16, 16)]
                s = srcb[pl.ds(v * 16, 16)]
                dl = d - lo
                msk = (dl >= 0) & (dl < NLOC)
                eid = lax.iota(jnp.int32, 16) + (g + v * 16)
                pos = plsc.cumsum(jnp.where(msk, 1, 0))
                dest = jnp.where(msk, (wv + pos - 1) & MASKC, TRASH)
                plsc.store_scatter(idc, [dest], eid)
                plsc.store_scatter(srcc, [dest], s)
                plsc.store_scatter(dlocc, [dest], dl)
                return wv + plsc.all_reduce_population_count(msk)
            wv1 = lax.fori_loop(0, NV, _filt,
                                jnp.full((16,), w0, jnp.int32))
            w1 = wv1[0]

            nwin = (w1 - r0) // SUB

            def _sub(k, r):
                _window(pl.multiple_of(r & MASKC, SUB), jnp.int32(SUB))
                return r + SUB
            r1 = lax.fori_loop(0, nwin, _sub, r0)
            return (w1, r1)
        w, r = lax.fori_loop(0, NCHUNK, _chunk,
                             (jnp.int32(0), jnp.int32(0)))

        def _drain():
            _window(pl.multiple_of(r & MASKC, SUB), w - r)
        lax.cond(w > r, _drain, lambda: None)

        pltpu.sync_copy(accs, ssum_hbm.at[pl.ds(lo * H, NLOC * H)])
        pltpu.sync_copy(accq, ssq_hbm.at[pl.ds(lo * H, NLOC * H)])
        pltpu.sync_copy(accmn, smn_hbm.at[pl.ds(lo * H, NLOC * H)])
        pltpu.sync_copy(accmx, smx_hbm.at[pl.ds(lo * H, NLOC * H)])
        pltpu.sync_copy(accd.at[pl.ds(0, NLOC)], deg_hbm.at[pl.ds(lo, NLOC)])


_edge_call = functools.partial(
    pl.kernel,
    out_type=[
        jax.ShapeDtypeStruct((NPAD * H,), jnp.float32),
        jax.ShapeDtypeStruct((NPAD * H,), jnp.float32),
        jax.ShapeDtypeStruct((NPAD * H,), jnp.float32),
        jax.ShapeDtypeStruct((NPAD * H,), jnp.float32),
        jax.ShapeDtypeStruct((NPAD,), jnp.float32),
    ],
    mesh=plsc.VectorSubcoreMesh(core_axis_name="c", subcore_axis_name="s"),
    compiler_params=pltpu.CompilerParams(needs_layout_passes=False),
    scratch_types=[
        pltpu.VMEM((CHUNK,), jnp.int32),
        pltpu.VMEM((CHUNK,), jnp.int32),
        pltpu.VMEM((CBUF,), jnp.int32),
        pltpu.VMEM((CBUF,), jnp.int32),
        pltpu.VMEM((CBUF,), jnp.int32),
        pltpu.VMEM((SUB, H), jnp.float32),
        pltpu.VMEM((SUB, H), jnp.float32),
        pltpu.VMEM((NLOC * H,), jnp.float32),
        pltpu.VMEM((NLOC * H,), jnp.float32),
        pltpu.VMEM((NLOC * H,), jnp.float32),
        pltpu.VMEM((NLOC * H,), jnp.float32),
        pltpu.VMEM((NLOC * H,), jnp.float32),
        pltpu.VMEM((NLOC + 16,), jnp.float32),
        pltpu.SemaphoreType.DMA,
        pltpu.SemaphoreType.DMA,
    ],
)(_edge_body)


RB = 400


def _ab_body(x_ref, w1_ref, w2_ref, a_ref, b_ref):
    xb = x_ref[...]
    a_ref[...] = jnp.dot(xb, w1_ref[...], preferred_element_type=jnp.float32)
    b_ref[...] = jnp.dot(xb, w2_ref[...], preferred_element_type=jnp.float32)


def _ab_call(x, w1, w2):
    return pl.pallas_call(
        _ab_body,
        grid=(N // RB,),
        in_specs=[
            pl.BlockSpec((RB, D), lambda i: (i, 0)),
            pl.BlockSpec((D, H), lambda i: (0, 0)),
            pl.BlockSpec((D, H), lambda i: (0, 0)),
        ],
        out_specs=[
            pl.BlockSpec((RB, H), lambda i: (i, 0)),
            pl.BlockSpec((RB, H), lambda i: (i, 0)),
        ],
        out_shape=[
            jax.ShapeDtypeStruct((N, H), jnp.float32),
            jax.ShapeDtypeStruct((N, H), jnp.float32),
        ],
    )(x, w1, w2)


EB = 8000


def _c_body(ea_ref, w3_ref, bp_ref, c_ref):
    c_ref[...] = (jnp.dot(ea_ref[...], w3_ref[...],
                          preferred_element_type=jnp.float32) + bp_ref[...])


def _c_call(ea, w3, bp):
    return pl.pallas_call(
        _c_body,
        grid=(E // EB,),
        in_specs=[
            pl.BlockSpec((EB, DE), lambda i: (i, 0)),
            pl.BlockSpec((DE, H), lambda i: (0, 0)),
            pl.BlockSpec((1, H), lambda i: (0, 0)),
        ],
        out_specs=pl.BlockSpec((EB, H), lambda i: (i, 0)),
        out_shape=jax.ShapeDtypeStruct((E, H), jnp.float32),
    )(ea, w3, bp)


def _stats_body(degb_ref, out_ref):
    col = degb_ref[:, 0:1]
    delta = jnp.sum(jnp.log(col + 1.0)) / N
    dmean = jnp.sum(col) / N
    rows = lax.broadcasted_iota(jnp.int32, (8, 128), 0)
    out_ref[...] = jnp.where(rows < 4, delta, dmean)


def _stats_call(degb):
    return pl.pallas_call(
        _stats_body,
        grid=(1,),
        in_specs=[pl.BlockSpec((N, H), lambda i: (0, 0))],
        out_specs=pl.BlockSpec((8, 128), lambda i: (0, 0)),
        out_shape=jax.ShapeDtypeStruct((8, 128), jnp.float32),
    )(degb)


def _post_body(ssum_ref, ssq_ref, smn_ref, smx_ref, degb_ref, x_ref,
               scal_ref, wp_ref, bp_ref, g_ref, b_ref, o_ref):
    dg = degb_ref[...]
    degc = jnp.maximum(dg, 1.0)
    mean = ssum_ref[...] / degc
    sq = ssq_ref[...] / degc
    std = jnp.sqrt(jnp.maximum(sq - mean * mean, 0.0) + 1e-5)
    pos = dg > 0.0
    mn = jnp.where(pos, smn_ref[...], 0.0)
    mx = jnp.where(pos, smx_ref[...], 0.0)
    delta = scal_ref[0, 0]
    dmean = scal_ref[1, 0]
    amp = jnp.log(dg + 1.0) / (delta + 1e-6)
    lin = dg / (dmean + 1e-6)

    out = jnp.broadcast_to(bp_ref[...], (RB, H))
    for k, t in enumerate((mean, mn, mx, std)):
        out = out + jnp.dot(t, wp_ref[k * H:(k + 1) * H, :],
                            preferred_element_type=jnp.float32)
        out = out + jnp.dot(t * amp, wp_ref[(4 + k) * H:(5 + k) * H, :],
                            preferred_element_type=jnp.float32)
        out = out + jnp.dot(t * lin, wp_ref[(8 + k) * H:(9 + k) * H, :],
                            preferred_element_type=jnp.float32)
    h = x_ref[...] + out
    mu = jnp.mean(h, axis=-1, keepdims=True)
    var = jnp.mean((h - mu) * (h - mu), axis=-1, keepdims=True)
    o_ref[...] = (h - mu) / jnp.sqrt(var + 1e-5) * g_ref[...] + b_ref[...]


def _post_call(ssum, ssq, smn, smx, degb, x, scal, wp, bp, g, b):
    blk = lambda i: (i, 0)
    return pl.pallas_call(
        _post_body,
        grid=(N // RB,),
        in_specs=[
            pl.BlockSpec((RB, H), blk),
            pl.BlockSpec((RB, H), blk),
            pl.BlockSpec((RB, H), blk),
            pl.BlockSpec((RB, H), blk),
            pl.BlockSpec((RB, H), blk),
            pl.BlockSpec((RB, D), blk),
            pl.BlockSpec(memory_space=pltpu.SMEM),
            pl.BlockSpec((12 * H, H), lambda i: (0, 0)),
            pl.BlockSpec((1, H), lambda i: (0, 0)),
            pl.BlockSpec((1, H), lambda i: (0, 0)),
            pl.BlockSpec((1, H), lambda i: (0, 0)),
        ],
        out_specs=pl.BlockSpec((RB, H), blk),
        out_shape=jax.ShapeDtypeStruct((N, H), jnp.float32),
    )(ssum, ssq, smn, smx, degb, x, scal, wp, bp, g, b)


def kernel(x, edge_index, edge_attr, W_pre, b_pre, W_post, b_post, gamma, beta):
    src = edge_index[0]
    dst = edge_index[1]
    w1 = W_pre[:D]
    w2 = W_pre[D:2 * D]
    w3 = W_pre[2 * D:]

    a, b = _ab_call(x, w1, w2)
    bflat = jnp.pad(b, ((0, NPAD - N), (0, 0))).reshape(-1)
    c = _c_call(edge_attr, w3, b_pre.reshape(1, H))

    ssum, ssq, smn, smx, deg = _edge_call(src, dst, a, bflat, c)
    ssum = ssum.reshape(NPAD, H)[:N]
    ssq = ssq.reshape(NPAD, H)[:N]
    smn = smn.reshape(NPAD, H)[:N]
    smx = smx.reshape(NPAD, H)[:N]
    degb = jnp.broadcast_to(deg[:N, None], (N, H))

    stats = _stats_call(degb)
    scal = jnp.stack([stats[0, 0], stats[4, 0]]).reshape(2, 1)

    return _post_call(ssum, ssq, smn, smx, degb, x, scal, W_post,
                      b_post.reshape(1, H), gamma.reshape(1, H),
                      beta.reshape(1, H))


# speedup vs baseline: 1.1526x; 1.1526x over previous
import functools

import jax
import jax.numpy as jnp
from jax import lax
from jax.experimental import pallas as pl
from jax.experimental.pallas import tpu as pltpu
from jax.experimental.pallas import tpu_sc as plsc

N = 10000
E = 320000
D = 128
DE = 16
H = 128

NSLOT = 64
NLOC = 160
NPAD = NSLOT * NLOC
CHUNK = 1280
NCHUNK = E // CHUNK
NV = CHUNK // 16
SUB = 64
CAP = 2048
MASKC = CAP - 1
TRASH = CAP
CBUF = CAP + 16
FBIG = 3.0e38


def _edge_body(src_hbm, dst_hbm, a_hbm, b_hbm, c_hbm,
               ssum_hbm, ssq_hbm, smn_hbm, smx_hbm, deg_hbm,
               dstb, srcb, dstb1, srcb1, idc, srcc, dlocc, bufa, bufc, bloc,
               accs, accq, accmn, accmx, accd, sem_a, sem_c,
               sem_d0, sem_s0, sem_d1, sem_s1):
    wid = lax.axis_index("s") * 2 + lax.axis_index("c")

    zero16 = jnp.zeros((16,), jnp.float32)
    pos16 = jnp.full((16,), FBIG, jnp.float32)
    neg16 = jnp.full((16,), -FBIG, jnp.float32)
    izero16 = jnp.zeros((16,), jnp.int32)

    def _initpad(t, c):
        idc[pl.ds(t * 16, 16)] = izero16
        srcc[pl.ds(t * 16, 16)] = izero16
        dlocc[pl.ds(t * 16, 16)] = izero16
        return c
    lax.fori_loop(0, CBUF // 16, _initpad, 0)

    for p in range(2):
        slot = p * 32 + wid
        lo = slot * NLOC

        def _initacc(t, c):
            o = pl.ds(t * 16, 16)
            accs[o] = zero16
            accq[o] = zero16
            accmn[o] = pos16
            accmx[o] = neg16
            return c
        lax.fori_loop(0, NLOC * H // 16, _initacc, 0)

        def _initd(t, c):
            accd[pl.ds(t * 16, 16)] = zero16
            return c
        lax.fori_loop(0, (NLOC + 16) // 16, _initd, 0)

        pltpu.sync_copy(b_hbm.at[pl.ds(lo * H, NLOC * H)], bloc)

        one16 = jnp.where(lax.iota(jnp.int32, 16) == 0, 1.0, 0.0)

        def _window(r0, n):
            cp_a = pltpu.async_copy(
                a_hbm.at[srcc.at[pl.ds(r0, SUB)]], bufa, sem_a)
            cp_c = pltpu.async_copy(
                c_hbm.at[idc.at[pl.ds(r0, SUB)]], bufc, sem_c)
            cp_a.wait()
            cp_c.wait()

            def _edge(i, c2):
                row = dlocc[pl.ds(r0 + i, 16)][0]
                accd[pl.ds(row, 16)] = accd[pl.ds(row, 16)] + one16
                rb = row * H
                for j in range(H // 16):
                    o = pl.ds(rb + j * 16, 16)
                    a = bufa[i, pl.ds(j * 16, 16)]
                    cc = bufc[i, pl.ds(j * 16, 16)]
                    b = bloc[pl.ds(rb + j * 16, 16)]
                    m = jnp.maximum(a + b + cc, 0.0)
                    accs[o] = accs[o] + m
                    accq[o] = accq[o] + m * m
                    accmn[o] = jnp.minimum(accmn[o], m)
                    accmx[o] = jnp.maximum(accmx[o], m)
                return c2
            lax.fori_loop(0, n, _edge, 0)

        bufs = ((dstb, srcb, sem_d0, sem_s0),
                (dstb1, srcb1, sem_d1, sem_s1))

        def _start(ci, p):
            db, sb, sd, ss = bufs[p]
            g = ci * CHUNK
            pltpu.async_copy(dst_hbm.at[pl.ds(g, CHUNK)], db, sd)
            pltpu.async_copy(src_hbm.at[pl.ds(g, CHUNK)], sb, ss)

        def _wait(ci, p):
            db, sb, sd, ss = bufs[p]
            g = ci * CHUNK
            pltpu.make_async_copy(dst_hbm.at[pl.ds(g, CHUNK)], db, sd).wait()
            pltpu.make_async_copy(src_hbm.at[pl.ds(g, CHUNK)], sb, ss).wait()

        _start(0, 0)

        def _chunk(ci, wr):
            w0, r0 = wr
            g = ci * CHUNK

            def _filt_on(db, sb):
                def _filt(v, wv):
                    d = db[pl.ds(v * 16, 16)]
                    s = sb[pl.ds(v * 16, 16)]
                    dl = d - lo
                    msk = (dl >= 0) & (dl < NLOC)
                    eid = lax.iota(jnp.int32, 16) + (g + v * 16)
                    pos = plsc.cumsum(jnp.where(msk, 1, 0))
                    dest = jnp.where(msk, (wv + pos - 1) & MASKC, TRASH)
                    plsc.store_scatter(idc, [dest], eid)
                    plsc.store_scatter(srcc, [dest], s)
                    plsc.store_scatter(dlocc, [dest], dl)
                    return wv + plsc.all_reduce_population_count(msk)
                return lax.fori_loop(0, NV, _filt,
                                     jnp.full((16,), w0, jnp.int32))

            def _run(p):
                def go():
                    _wait(ci, p)
                    lax.cond(ci + 1 < NCHUNK,
                             lambda: _start(ci + 1, 1 - p), lambda: None)
                    return _filt_on(*bufs[p][:2])
                return go
            wv1 = lax.cond((ci & 1) == 0, _run(0), _run(1))
            w1 = wv1[0]

            nwin = (w1 - r0) // SUB

            def _sub(k, r):
                _window(pl.multiple_of(r & MASKC, SUB), jnp.int32(SUB))
                return r + SUB
            r1 = lax.fori_loop(0, nwin, _sub, r0)
            return (w1, r1)
        w, r = lax.fori_loop(0, NCHUNK, _chunk,
                             (jnp.int32(0), jnp.int32(0)))

        def _drain():
            _window(pl.multiple_of(r & MASKC, SUB), w - r)
        lax.cond(w > r, _drain, lambda: None)

        pltpu.sync_copy(accs, ssum_hbm.at[pl.ds(lo * H, NLOC * H)])
        pltpu.sync_copy(accq, ssq_hbm.at[pl.ds(lo * H, NLOC * H)])
        pltpu.sync_copy(accmn, smn_hbm.at[pl.ds(lo * H, NLOC * H)])
        pltpu.sync_copy(accmx, smx_hbm.at[pl.ds(lo * H, NLOC * H)])
        pltpu.sync_copy(accd.at[pl.ds(0, NLOC)], deg_hbm.at[pl.ds(lo, NLOC)])


_edge_call = functools.partial(
    pl.kernel,
    out_type=[
        jax.ShapeDtypeStruct((NPAD * H,), jnp.float32),
        jax.ShapeDtypeStruct((NPAD * H,), jnp.float32),
        jax.ShapeDtypeStruct((NPAD * H,), jnp.float32),
        jax.ShapeDtypeStruct((NPAD * H,), jnp.float32),
        jax.ShapeDtypeStruct((NPAD,), jnp.float32),
    ],
    mesh=plsc.VectorSubcoreMesh(core_axis_name="c", subcore_axis_name="s"),
    compiler_params=pltpu.CompilerParams(needs_layout_passes=False),
    scratch_types=[
        pltpu.VMEM((CHUNK,), jnp.int32),
        pltpu.VMEM((CHUNK,), jnp.int32),
        pltpu.VMEM((CHUNK,), jnp.int32),
        pltpu.VMEM((CHUNK,), jnp.int32),
        pltpu.VMEM((CBUF,), jnp.int32),
        pltpu.VMEM((CBUF,), jnp.int32),
        pltpu.VMEM((CBUF,), jnp.int32),
        pltpu.VMEM((SUB, H), jnp.float32),
        pltpu.VMEM((SUB, H), jnp.float32),
        pltpu.VMEM((NLOC * H,), jnp.float32),
        pltpu.VMEM((NLOC * H,), jnp.float32),
        pltpu.VMEM((NLOC * H,), jnp.float32),
        pltpu.VMEM((NLOC * H,), jnp.float32),
        pltpu.VMEM((NLOC * H,), jnp.float32),
        pltpu.VMEM((NLOC + 16,), jnp.float32),
        pltpu.SemaphoreType.DMA,
        pltpu.SemaphoreType.DMA,
        pltpu.SemaphoreType.DMA,
        pltpu.SemaphoreType.DMA,
        pltpu.SemaphoreType.DMA,
        pltpu.SemaphoreType.DMA,
    ],
)(_edge_body)


RB = 400


def _ab_body(x_ref, w1_ref, w2_ref, a_ref, b_ref):
    xb = x_ref[...]
    a_ref[...] = jnp.dot(xb, w1_ref[...], preferred_element_type=jnp.float32)
    b_ref[...] = jnp.dot(xb, w2_ref[...], preferred_element_type=jnp.float32)


def _ab_call(x, w1, w2):
    return pl.pallas_call(
        _ab_body,
        grid=(N // RB,),
        in_specs=[
            pl.BlockSpec((RB, D), lambda i: (i, 0)),
            pl.BlockSpec((D, H), lambda i: (0, 0)),
            pl.BlockSpec((D, H), lambda i: (0, 0)),
        ],
        out_specs=[
            pl.BlockSpec((RB, H), lambda i: (i, 0)),
            pl.BlockSpec((RB, H), lambda i: (i, 0)),
        ],
        out_shape=[
            jax.ShapeDtypeStruct((N, H), jnp.float32),
            jax.ShapeDtypeStruct((N, H), jnp.float32),
        ],
    )(x, w1, w2)


EB = 8000


def _c_body(ea_ref, w3_ref, bp_ref, c_ref):
    c_ref[...] = (jnp.dot(ea_ref[...], w3_ref[...],
                          preferred_element_type=jnp.float32) + bp_ref[...])


def _c_call(ea, w3, bp):
    return pl.pallas_call(
        _c_body,
        grid=(E // EB,),
        in_specs=[
            pl.BlockSpec((EB, DE), lambda i: (i, 0)),
            pl.BlockSpec((DE, H), lambda i: (0, 0)),
            pl.BlockSpec((1, H), lambda i: (0, 0)),
        ],
        out_specs=pl.BlockSpec((EB, H), lambda i: (i, 0)),
        out_shape=jax.ShapeDtypeStruct((E, H), jnp.float32),
    )(ea, w3, bp)


def _stats_body(degb_ref, out_ref):
    col = degb_ref[:, 0:1]
    delta = jnp.sum(jnp.log(col + 1.0)) / N
    dmean = jnp.sum(col) / N
    rows = lax.broadcasted_iota(jnp.int32, (8, 128), 0)
    out_ref[...] = jnp.where(rows < 4, delta, dmean)


def _stats_call(degb):
    return pl.pallas_call(
        _stats_body,
        grid=(1,),
        in_specs=[pl.BlockSpec((N, H), lambda i: (0, 0))],
        out_specs=pl.BlockSpec((8, 128), lambda i: (0, 0)),
        out_shape=jax.ShapeDtypeStruct((8, 128), jnp.float32),
    )(degb)


def _post_body(ssum_ref, ssq_ref, smn_ref, smx_ref, degb_ref, x_ref,
               scal_ref, wp_ref, bp_ref, g_ref, b_ref, o_ref):
    dg = degb_ref[...]
    degc = jnp.maximum(dg, 1.0)
    mean = ssum_ref[...] / degc
    sq = ssq_ref[...] / degc
    std = jnp.sqrt(jnp.maximum(sq - mean * mean, 0.0) + 1e-5)
    pos = dg > 0.0
    mn = jnp.where(pos, smn_ref[...], 0.0)
    mx = jnp.where(pos, smx_ref[...], 0.0)
    delta = scal_ref[0, 0]
    dmean = scal_ref[1, 0]
    amp = jnp.log(dg + 1.0) / (delta + 1e-6)
    lin = dg / (dmean + 1e-6)

    out = jnp.broadcast_to(bp_ref[...], (RB, H))
    for k, t in enumerate((mean, mn, mx, std)):
        out = out + jnp.dot(t, wp_ref[k * H:(k + 1) * H, :],
                            preferred_element_type=jnp.float32)
        out = out + jnp.dot(t * amp, wp_ref[(4 + k) * H:(5 + k) * H, :],
                            preferred_element_type=jnp.float32)
        out = out + jnp.dot(t * lin, wp_ref[(8 + k) * H:(9 + k) * H, :],
                            preferred_element_type=jnp.float32)
    h = x_ref[...] + out
    mu = jnp.mean(h, axis=-1, keepdims=True)
    var = jnp.mean((h - mu) * (h - mu), axis=-1, keepdims=True)
    o_ref[...] = (h - mu) / jnp.sqrt(var + 1e-5) * g_ref[...] + b_ref[...]


def _post_call(ssum, ssq, smn, smx, degb, x, scal, wp, bp, g, b):
    blk = lambda i: (i, 0)
    return pl.pallas_call(
        _post_body,
        grid=(N // RB,),
        in_specs=[
            pl.BlockSpec((RB, H), blk),
            pl.BlockSpec((RB, H), blk),
            pl.BlockSpec((RB, H), blk),
            pl.BlockSpec((RB, H), blk),
            pl.BlockSpec((RB, H), blk),
            pl.BlockSpec((RB, D), blk),
            pl.BlockSpec(memory_space=pltpu.SMEM),
            pl.BlockSpec((12 * H, H), lambda i: (0, 0)),
            pl.BlockSpec((1, H), lambda i: (0, 0)),
            pl.BlockSpec((1, H), lambda i: (0, 0)),
            pl.BlockSpec((1, H), lambda i: (0, 0)),
        ],
        out_specs=pl.BlockSpec((RB, H), blk),
        out_shape=jax.ShapeDtypeStruct((N, H), jnp.float32),
    )(ssum, ssq, smn, smx, degb, x, scal, wp, bp, g, b)


def kernel(x, edge_index, edge_attr, W_pre, b_pre, W_post, b_post, gamma, beta):
    src = edge_index[0]
    dst = edge_index[1]
    w1 = W_pre[:D]
    w2 = W_pre[D:2 * D]
    w3 = W_pre[2 * D:]

    a, b = _ab_call(x, w1, w2)
    bflat = jnp.pad(b, ((0, NPAD - N), (0, 0))).reshape(-1)
    c = _c_call(edge_attr, w3, b_pre.reshape(1, H))

    ssum, ssq, smn, smx, deg = _edge_call(src, dst, a, bflat, c)
    ssum = ssum.reshape(NPAD, H)[:N]
    ssq = ssq.reshape(NPAD, H)[:N]
    smn = smn.reshape(NPAD, H)[:N]
    smx = smx.reshape(NPAD, H)[:N]
    degb = jnp.broadcast_to(deg[:N, None], (N, H))

    stats = _stats_call(degb)
    scal = jnp.stack([stats[0, 0], stats[4, 0]]).reshape(2, 1)

    return _post_call(ssum, ssq, smn, smx, degb, x, scal, W_post,
                      b_post.reshape(1, H), gamma.reshape(1, H),
                      beta.reshape(1, H))


# speedup vs baseline: 1.2591x; 1.0924x over previous
import functools

import jax
import jax.numpy as jnp
from jax import lax
from jax.experimental import pallas as pl
from jax.experimental.pallas import tpu as pltpu
from jax.experimental.pallas import tpu_sc as plsc

N = 10000
E = 320000
D = 128
DE = 16
H = 128

NSLOT = 64
NLOC = 160
NPAD = NSLOT * NLOC
CHUNK = 1280
NCHUNK = E // CHUNK
NV = CHUNK // 16
SUB = 64
CAP = 2048
MASKC = CAP - 1
TRASH = CAP
CBUF = CAP + 16
FBIG = 3.0e38


def _edge_body(src_hbm, dst_hbm, a_hbm, b_hbm, c_hbm,
               ssum_hbm, ssq_hbm, smn_hbm, smx_hbm, deg_hbm,
               dstb, srcb, dstb1, srcb1, idc, srcc, dlocc, bufa, bufc, bloc,
               accs, accq, accmn, accmx, accd, sem_a, sem_c,
               sem_d0, sem_s0, sem_d1, sem_s1):
    wid = lax.axis_index("s") * 2 + lax.axis_index("c")

    zero16 = jnp.zeros((16,), jnp.float32)
    pos16 = jnp.full((16,), FBIG, jnp.float32)
    neg16 = jnp.full((16,), -FBIG, jnp.float32)
    izero16 = jnp.zeros((16,), jnp.int32)

    def _initpad(t, c):
        idc[pl.ds(t * 16, 16)] = izero16
        srcc[pl.ds(t * 16, 16)] = izero16
        dlocc[pl.ds(t * 16, 16)] = izero16
        return c
    lax.fori_loop(0, CBUF // 16, _initpad, 0)

    for p in range(2):
        slot = p * 32 + wid
        lo = slot * NLOC

        def _initacc(t, c):
            o = pl.ds(t * 16, 16)
            accs[o] = zero16
            accq[o] = zero16
            accmn[o] = pos16
            accmx[o] = neg16
            return c
        lax.fori_loop(0, NLOC * H // 16, _initacc, 0)

        def _initd(t, c):
            accd[pl.ds(t * 16, 16)] = zero16
            return c
        lax.fori_loop(0, (NLOC + 16) // 16, _initd, 0)

        pltpu.sync_copy(b_hbm.at[pl.ds(lo * H, NLOC * H)], bloc)

        one16 = jnp.where(lax.iota(jnp.int32, 16) == 0, 1.0, 0.0)

        def _fire(r0):
            pltpu.async_copy(a_hbm.at[srcc.at[pl.ds(r0, SUB)]], bufa, sem_a)
            pltpu.async_copy(c_hbm.at[idc.at[pl.ds(r0, SUB)]], bufc, sem_c)

        def _acc(r0, n):
            pltpu.make_async_copy(
                a_hbm.at[srcc.at[pl.ds(r0, SUB)]], bufa, sem_a).wait()
            pltpu.make_async_copy(
                c_hbm.at[idc.at[pl.ds(r0, SUB)]], bufc, sem_c).wait()

            def _edge(i, c2):
                row = dlocc[pl.ds(r0 + i, 16)][0]
                accd[pl.ds(row, 16)] = accd[pl.ds(row, 16)] + one16
                rb = row * H
                for j in range(H // 16):
                    o = pl.ds(rb + j * 16, 16)
                    a = bufa[i, pl.ds(j * 16, 16)]
                    cc = bufc[i, pl.ds(j * 16, 16)]
                    b = bloc[pl.ds(rb + j * 16, 16)]
                    m = jnp.maximum(a + b + cc, 0.0)
                    accs[o] = accs[o] + m
                    accq[o] = accq[o] + m * m
                    accmn[o] = jnp.minimum(accmn[o], m)
                    accmx[o] = jnp.maximum(accmx[o], m)
                return c2
            lax.fori_loop(0, n, _edge, 0)

        def _window(r0, n):
            _fire(r0)
            _acc(r0, n)

        def _rmask(r):
            return pl.multiple_of(r & MASKC, SUB)

        bufs = ((dstb, srcb, sem_d0, sem_s0),
                (dstb1, srcb1, sem_d1, sem_s1))

        def _start(ci, p):
            db, sb, sd, ss = bufs[p]
            g = ci * CHUNK
            pltpu.async_copy(dst_hbm.at[pl.ds(g, CHUNK)], db, sd)
            pltpu.async_copy(src_hbm.at[pl.ds(g, CHUNK)], sb, ss)

        def _wait(ci, p):
            db, sb, sd, ss = bufs[p]
            g = ci * CHUNK
            pltpu.make_async_copy(dst_hbm.at[pl.ds(g, CHUNK)], db, sd).wait()
            pltpu.make_async_copy(src_hbm.at[pl.ds(g, CHUNK)], sb, ss).wait()

        _start(0, 0)

        def _chunk(ci, wrp):
            w0, r0, pend = wrp
            g = ci * CHUNK

            def _filt_on(db, sb):
                def _filt(v, wv):
                    d = db[pl.ds(v * 16, 16)]
                    s = sb[pl.ds(v * 16, 16)]
                    dl = d - lo
                    msk = (dl >= 0) & (dl < NLOC)
                    eid = lax.iota(jnp.int32, 16) + (g + v * 16)
                    pos = plsc.cumsum(jnp.where(msk, 1, 0))
                    dest = jnp.where(msk, (wv + pos - 1) & MASKC, TRASH)
                    plsc.store_scatter(idc, [dest], eid)
                    plsc.store_scatter(srcc, [dest], s)
                    plsc.store_scatter(dlocc, [dest], dl)
                    return wv + plsc.all_reduce_population_count(msk)
                return lax.fori_loop(0, NV, _filt,
                                     jnp.full((16,), w0, jnp.int32))

            def _run(p):
                def go():
                    _wait(ci, p)
                    lax.cond(ci + 1 < NCHUNK,
                             lambda: _start(ci + 1, 1 - p), lambda: None)
                    return _filt_on(*bufs[p][:2])
                return go
            wv1 = lax.cond((ci & 1) == 0, _run(0), _run(1))
            w1 = wv1[0]

            r1 = lax.cond(
                pend == 1,
                lambda: (_acc(_rmask(r0), jnp.int32(SUB)), r0 + SUB)[1],
                lambda: r0)

            nwin = (w1 - r1) // SUB

            def _sub(k, r):
                _window(_rmask(r), jnp.int32(SUB))
                return r + SUB
            r2 = lax.fori_loop(0, jnp.maximum(nwin - 1, 0), _sub, r1)
            pend1 = lax.cond(
                nwin >= 1,
                lambda: (_fire(_rmask(r2)), jnp.int32(1))[1],
                lambda: jnp.int32(0))
            return (w1, r2, pend1)
        w, r, pend = lax.fori_loop(
            0, NCHUNK, _chunk,
            (jnp.int32(0), jnp.int32(0), jnp.int32(0)))

        r = lax.cond(
            pend == 1,
            lambda: (_acc(_rmask(r), jnp.int32(SUB)), r + SUB)[1],
            lambda: r)

        def _drain():
            _window(_rmask(r), w - r)
        lax.cond(w > r, _drain, lambda: None)

        pltpu.sync_copy(accs, ssum_hbm.at[pl.ds(lo * H, NLOC * H)])
        pltpu.sync_copy(accq, ssq_hbm.at[pl.ds(lo * H, NLOC * H)])
        pltpu.sync_copy(accmn, smn_hbm.at[pl.ds(lo * H, NLOC * H)])
        pltpu.sync_copy(accmx, smx_hbm.at[pl.ds(lo * H, NLOC * H)])
        pltpu.sync_copy(accd.at[pl.ds(0, NLOC)], deg_hbm.at[pl.ds(lo, NLOC)])


_edge_call = functools.partial(
    pl.kernel,
    out_type=[
        jax.ShapeDtypeStruct((NPAD * H,), jnp.float32),
        jax.ShapeDtypeStruct((NPAD * H,), jnp.float32),
        jax.ShapeDtypeStruct((NPAD * H,), jnp.float32),
        jax.ShapeDtypeStruct((NPAD * H,), jnp.float32),
        jax.ShapeDtypeStruct((NPAD,), jnp.float32),
    ],
    mesh=plsc.VectorSubcoreMesh(core_axis_name="c", subcore_axis_name="s"),
    compiler_params=pltpu.CompilerParams(needs_layout_passes=False),
    scratch_types=[
        pltpu.VMEM((CHUNK,), jnp.int32),
        pltpu.VMEM((CHUNK,), jnp.int32),
        pltpu.VMEM((CHUNK,), jnp.int32),
        pltpu.VMEM((CHUNK,), jnp.int32),
        pltpu.VMEM((CBUF,), jnp.int32),
        pltpu.VMEM((CBUF,), jnp.int32),
        pltpu.VMEM((CBUF,), jnp.int32),
        pltpu.VMEM((SUB, H), jnp.float32),
        pltpu.VMEM((SUB, H), jnp.float32),
        pltpu.VMEM((NLOC * H,), jnp.float32),
        pltpu.VMEM((NLOC * H,), jnp.float32),
        pltpu.VMEM((NLOC * H,), jnp.float32),
        pltpu.VMEM((NLOC * H,), jnp.float32),
        pltpu.VMEM((NLOC * H,), jnp.float32),
        pltpu.VMEM((NLOC + 16,), jnp.float32),
        pltpu.SemaphoreType.DMA,
        pltpu.SemaphoreType.DMA,
        pltpu.SemaphoreType.DMA,
        pltpu.SemaphoreType.DMA,
        pltpu.SemaphoreType.DMA,
        pltpu.SemaphoreType.DMA,
    ],
)(_edge_body)


RB = 400


def _ab_body(x_ref, w1_ref, w2_ref, a_ref, b_ref):
    xb = x_ref[...]
    a_ref[...] = jnp.dot(xb, w1_ref[...], preferred_element_type=jnp.float32)
    b_ref[...] = jnp.dot(xb, w2_ref[...], preferred_element_type=jnp.float32)


def _ab_call(x, w1, w2):
    return pl.pallas_call(
        _ab_body,
        grid=(N // RB,),
        in_specs=[
            pl.BlockSpec((RB, D), lambda i: (i, 0)),
            pl.BlockSpec((D, H), lambda i: (0, 0)),
            pl.BlockSpec((D, H), lambda i: (0, 0)),
        ],
        out_specs=[
            pl.BlockSpec((RB, H), lambda i: (i, 0)),
            pl.BlockSpec((RB, H), lambda i: (i, 0)),
        ],
        out_shape=[
            jax.ShapeDtypeStruct((N, H), jnp.float32),
            jax.ShapeDtypeStruct((N, H), jnp.float32),
        ],
    )(x, w1, w2)


EB = 8000


def _c_body(ea_ref, w3_ref, bp_ref, c_ref):
    c_ref[...] = (jnp.dot(ea_ref[...], w3_ref[...],
                          preferred_element_type=jnp.float32) + bp_ref[...])


def _c_call(ea, w3, bp):
    return pl.pallas_call(
        _c_body,
        grid=(E // EB,),
        in_specs=[
            pl.BlockSpec((EB, DE), lambda i: (i, 0)),
            pl.BlockSpec((DE, H), lambda i: (0, 0)),
            pl.BlockSpec((1, H), lambda i: (0, 0)),
        ],
        out_specs=pl.BlockSpec((EB, H), lambda i: (i, 0)),
        out_shape=jax.ShapeDtypeStruct((E, H), jnp.float32),
    )(ea, w3, bp)


def _stats_body(degb_ref, out_ref):
    col = degb_ref[:, 0:1]
    delta = jnp.sum(jnp.log(col + 1.0)) / N
    dmean = jnp.sum(col) / N
    rows = lax.broadcasted_iota(jnp.int32, (8, 128), 0)
    out_ref[...] = jnp.where(rows < 4, delta, dmean)


def _stats_call(degb):
    return pl.pallas_call(
        _stats_body,
        grid=(1,),
        in_specs=[pl.BlockSpec((N, H), lambda i: (0, 0))],
        out_specs=pl.BlockSpec((8, 128), lambda i: (0, 0)),
        out_shape=jax.ShapeDtypeStruct((8, 128), jnp.float32),
    )(degb)


def _post_body(ssum_ref, ssq_ref, smn_ref, smx_ref, degb_ref, x_ref,
               scal_ref, wp_ref, bp_ref, g_ref, b_ref, o_ref):
    dg = degb_ref[...]
    degc = jnp.maximum(dg, 1.0)
    mean = ssum_ref[...] / degc
    sq = ssq_ref[...] / degc
    std = jnp.sqrt(jnp.maximum(sq - mean * mean, 0.0) + 1e-5)
    pos = dg > 0.0
    mn = jnp.where(pos, smn_ref[...], 0.0)
    mx = jnp.where(pos, smx_ref[...], 0.0)
    delta = scal_ref[0, 0]
    dmean = scal_ref[1, 0]
    amp = jnp.log(dg + 1.0) / (delta + 1e-6)
    lin = dg / (dmean + 1e-6)

    out = jnp.broadcast_to(bp_ref[...], (RB, H))
    for k, t in enumerate((mean, mn, mx, std)):
        out = out + jnp.dot(t, wp_ref[k * H:(k + 1) * H, :],
                            preferred_element_type=jnp.float32)
        out = out + jnp.dot(t * amp, wp_ref[(4 + k) * H:(5 + k) * H, :],
                            preferred_element_type=jnp.float32)
        out = out + jnp.dot(t * lin, wp_ref[(8 + k) * H:(9 + k) * H, :],
                            preferred_element_type=jnp.float32)
    h = x_ref[...] + out
    mu = jnp.mean(h, axis=-1, keepdims=True)
    var = jnp.mean((h - mu) * (h - mu), axis=-1, keepdims=True)
    o_ref[...] = (h - mu) / jnp.sqrt(var + 1e-5) * g_ref[...] + b_ref[...]


def _post_call(ssum, ssq, smn, smx, degb, x, scal, wp, bp, g, b):
    blk = lambda i: (i, 0)
    return pl.pallas_call(
        _post_body,
        grid=(N // RB,),
        in_specs=[
            pl.BlockSpec((RB, H), blk),
            pl.BlockSpec((RB, H), blk),
            pl.BlockSpec((RB, H), blk),
            pl.BlockSpec((RB, H), blk),
            pl.BlockSpec((RB, H), blk),
            pl.BlockSpec((RB, D), blk),
            pl.BlockSpec(memory_space=pltpu.SMEM),
            pl.BlockSpec((12 * H, H), lambda i: (0, 0)),
            pl.BlockSpec((1, H), lambda i: (0, 0)),
            pl.BlockSpec((1, H), lambda i: (0, 0)),
            pl.BlockSpec((1, H), lambda i: (0, 0)),
        ],
        out_specs=pl.BlockSpec((RB, H), blk),
        out_shape=jax.ShapeDtypeStruct((N, H), jnp.float32),
    )(ssum, ssq, smn, smx, degb, x, scal, wp, bp, g, b)


def kernel(x, edge_index, edge_attr, W_pre, b_pre, W_post, b_post, gamma, beta):
    src = edge_index[0]
    dst = edge_index[1]
    w1 = W_pre[:D]
    w2 = W_pre[D:2 * D]
    w3 = W_pre[2 * D:]

    a, b = _ab_call(x, w1, w2)
    bflat = jnp.pad(b, ((0, NPAD - N), (0, 0))).reshape(-1)
    c = _c_call(edge_attr, w3, b_pre.reshape(1, H))

    ssum, ssq, smn, smx, deg = _edge_call(src, dst, a, bflat, c)
    ssum = ssum.reshape(NPAD, H)[:N]
    ssq = ssq.reshape(NPAD, H)[:N]
    smn = smn.reshape(NPAD, H)[:N]
    smx = smx.reshape(NPAD, H)[:N]
    degb = jnp.broadcast_to(deg[:N, None], (N, H))

    stats = _stats_call(degb)
    scal = jnp.stack([stats[0, 0], stats[4, 0]]).reshape(2, 1)

    return _post_call(ssum, ssq, smn, smx, degb, x, scal, W_post,
                      b_post.reshape(1, H), gamma.reshape(1, H),
                      beta.reshape(1, H))


# speedup vs baseline: 1.2715x; 1.0099x over previous
import functools

import jax
import jax.numpy as jnp
from jax import lax
from jax.experimental import pallas as pl
from jax.experimental.pallas import tpu as pltpu
from jax.experimental.pallas import tpu_sc as plsc

N = 10000
E = 320000
D = 128
DE = 16
H = 128

NSLOT = 64
NLOC = 160
NPAD = NSLOT * NLOC
CHUNK = 1280
NCHUNK = E // CHUNK
NV = CHUNK // 16
SUB = 64
CAP = 2048
MASKC = CAP - 1
TRASH = CAP
CBUF = CAP + 16
FBIG = 3.0e38


def _edge_body(src_hbm, dst_hbm, a_hbm, b_hbm, c_hbm,
               ssum_hbm, ssq_hbm, smn_hbm, smx_hbm, deg_hbm,
               dstb, srcb, dstb1, srcb1, idc, srcc, dlocc, bufa, bufc, bloc,
               accs, accq, accmn, accmx, accd, sem_a, sem_c,
               sem_d0, sem_s0, sem_d1, sem_s1):
    wid = lax.axis_index("s") * 2 + lax.axis_index("c")

    zero16 = jnp.zeros((16,), jnp.float32)
    pos16 = jnp.full((16,), FBIG, jnp.float32)
    neg16 = jnp.full((16,), -FBIG, jnp.float32)
    izero16 = jnp.zeros((16,), jnp.int32)

    def _initpad(t, c):
        idc[pl.ds(t * 16, 16)] = izero16
        srcc[pl.ds(t * 16, 16)] = izero16
        dlocc[pl.ds(t * 16, 16)] = izero16
        return c
    lax.fori_loop(0, CBUF // 16, _initpad, 0)

    for p in range(2):
        slot = p * 32 + wid
        lo = slot * NLOC

        def _initacc(t, c):
            o = pl.ds(t * 16, 16)
            accs[o] = zero16
            accq[o] = zero16
            accmn[o] = pos16
            accmx[o] = neg16
            return c
        lax.fori_loop(0, NLOC * H // 16, _initacc, 0)

        def _initd(t, c):
            accd[pl.ds(t * 16, 16)] = zero16
            return c
        lax.fori_loop(0, (NLOC + 16) // 16, _initd, 0)

        pltpu.sync_copy(b_hbm.at[pl.ds(lo * H, NLOC * H)], bloc)

        one16 = jnp.where(lax.iota(jnp.int32, 16) == 0, 1.0, 0.0)

        def _fire(r0):
            pltpu.async_copy(a_hbm.at[srcc.at[pl.ds(r0, SUB)]], bufa, sem_a)
            pltpu.async_copy(c_hbm.at[idc.at[pl.ds(r0, SUB)]], bufc, sem_c)

        def _acc(r0, n):
            pltpu.make_async_copy(
                a_hbm.at[srcc.at[pl.ds(r0, SUB)]], bufa, sem_a).wait()
            pltpu.make_async_copy(
                c_hbm.at[idc.at[pl.ds(r0, SUB)]], bufc, sem_c).wait()

            def _edge(i, c2):
                row = dlocc[pl.ds(r0 + i, 16)][0]
                accd[pl.ds(row, 16)] = accd[pl.ds(row, 16)] + one16
                rb = row * H
                for j in range(H // 16):
                    o = pl.ds(rb + j * 16, 16)
                    a = bufa[i, pl.ds(j * 16, 16)]
                    cc = bufc[i, pl.ds(j * 16, 16)]
                    b = bloc[pl.ds(rb + j * 16, 16)]
                    m = jnp.maximum(a + b + cc, 0.0)
                    accs[o] = accs[o] + m
                    accq[o] = accq[o] + m * m
                    accmn[o] = jnp.minimum(accmn[o], m)
                    accmx[o] = jnp.maximum(accmx[o], m)
                return c2
            lax.fori_loop(0, n, _edge, 0)

        def _window(r0, n):
            _fire(r0)
            _acc(r0, n)

        def _rmask(r):
            return pl.multiple_of(r & MASKC, SUB)

        bufs = ((dstb, srcb, sem_d0, sem_s0),
                (dstb1, srcb1, sem_d1, sem_s1))

        def _start(ci, p):
            db, sb, sd, ss = bufs[p]
            g = ci * CHUNK
            pltpu.async_copy(dst_hbm.at[pl.ds(g, CHUNK)], db, sd)
            pltpu.async_copy(src_hbm.at[pl.ds(g, CHUNK)], sb, ss)

        def _wait(ci, p):
            db, sb, sd, ss = bufs[p]
            g = ci * CHUNK
            pltpu.make_async_copy(dst_hbm.at[pl.ds(g, CHUNK)], db, sd).wait()
            pltpu.make_async_copy(src_hbm.at[pl.ds(g, CHUNK)], sb, ss).wait()

        _start(0, 0)

        def _chunk(ci, wrp):
            w0, r0, pend = wrp
            g = ci * CHUNK

            def _filt_on(db, sb):
                def _filt(v, wv):
                    d = db[pl.ds(v * 16, 16)]
                    s = sb[pl.ds(v * 16, 16)]
                    dl = d - lo
                    msk = (dl >= 0) & (dl < NLOC)
                    eid = lax.iota(jnp.int32, 16) + (g + v * 16)
                    pos = plsc.cumsum(jnp.where(msk, 1, 0))
                    dest = jnp.where(msk, (wv + pos - 1) & MASKC, TRASH)
                    plsc.store_scatter(idc, [dest], eid)
                    plsc.store_scatter(srcc, [dest], s)
                    plsc.store_scatter(dlocc, [dest], dl)
                    return wv + plsc.all_reduce_population_count(msk)
                return lax.fori_loop(0, NV, _filt,
                                     jnp.full((16,), w0, jnp.int32))

            def _run(p):
                def go():
                    _wait(ci, p)
                    lax.cond(ci + 1 < NCHUNK,
                             lambda: _start(ci + 1, 1 - p), lambda: None)
                    return _filt_on(*bufs[p][:2])
                return go
            wv1 = lax.cond((ci & 1) == 0, _run(0), _run(1))
            w1 = wv1[0]

            r1 = lax.cond(
                pend == 1,
                lambda: (_acc(_rmask(r0), jnp.int32(SUB)), r0 + SUB)[1],
                lambda: r0)

            nwin = (w1 - r1) // SUB

            def _sub(k, r):
                _window(_rmask(r), jnp.int32(SUB))
                return r + SUB
            r2 = lax.fori_loop(0, jnp.maximum(nwin - 1, 0), _sub, r1)
            pend1 = lax.cond(
                nwin >= 1,
                lambda: (_fire(_rmask(r2)), jnp.int32(1))[1],
                lambda: jnp.int32(0))
            return (w1, r2, pend1)
        w, r, pend = lax.fori_loop(
            0, NCHUNK, _chunk,
            (jnp.int32(0), jnp.int32(0), jnp.int32(0)))

        r = lax.cond(
            pend == 1,
            lambda: (_acc(_rmask(r), jnp.int32(SUB)), r + SUB)[1],
            lambda: r)

        def _drain():
            _window(_rmask(r), w - r)
        lax.cond(w > r, _drain, lambda: None)

        pltpu.sync_copy(accs, ssum_hbm.at[pl.ds(lo * H, NLOC * H)])
        pltpu.sync_copy(accq, ssq_hbm.at[pl.ds(lo * H, NLOC * H)])
        pltpu.sync_copy(accmn, smn_hbm.at[pl.ds(lo * H, NLOC * H)])
        pltpu.sync_copy(accmx, smx_hbm.at[pl.ds(lo * H, NLOC * H)])
        pltpu.sync_copy(accd.at[pl.ds(0, NLOC)], deg_hbm.at[pl.ds(lo, NLOC)])


_edge_call = functools.partial(
    pl.kernel,
    out_type=[
        jax.ShapeDtypeStruct((NPAD * H,), jnp.float32),
        jax.ShapeDtypeStruct((NPAD * H,), jnp.float32),
        jax.ShapeDtypeStruct((NPAD * H,), jnp.float32),
        jax.ShapeDtypeStruct((NPAD * H,), jnp.float32),
        jax.ShapeDtypeStruct((NPAD,), jnp.float32),
    ],
    mesh=plsc.VectorSubcoreMesh(core_axis_name="c", subcore_axis_name="s"),
    compiler_params=pltpu.CompilerParams(needs_layout_passes=False),
    scratch_types=[
        pltpu.VMEM((CHUNK,), jnp.int32),
        pltpu.VMEM((CHUNK,), jnp.int32),
        pltpu.VMEM((CHUNK,), jnp.int32),
        pltpu.VMEM((CHUNK,), jnp.int32),
        pltpu.VMEM((CBUF,), jnp.int32),
        pltpu.VMEM((CBUF,), jnp.int32),
        pltpu.VMEM((CBUF,), jnp.int32),
        pltpu.VMEM((SUB, H), jnp.float32),
        pltpu.VMEM((SUB, H), jnp.float32),
        pltpu.VMEM((NLOC * H,), jnp.float32),
        pltpu.VMEM((NLOC * H,), jnp.float32),
        pltpu.VMEM((NLOC * H,), jnp.float32),
        pltpu.VMEM((NLOC * H,), jnp.float32),
        pltpu.VMEM((NLOC * H,), jnp.float32),
        pltpu.VMEM((NLOC + 16,), jnp.float32),
        pltpu.SemaphoreType.DMA,
        pltpu.SemaphoreType.DMA,
        pltpu.SemaphoreType.DMA,
        pltpu.SemaphoreType.DMA,
        pltpu.SemaphoreType.DMA,
        pltpu.SemaphoreType.DMA,
    ],
)(_edge_body)


RB = 400


def _ab_body(x_ref, w1_ref, w2_ref, a_ref, b_ref):
    xb = x_ref[...]
    a_ref[...] = jnp.dot(xb, w1_ref[...], preferred_element_type=jnp.float32)
    b_ref[...] = jnp.dot(xb, w2_ref[...], preferred_element_type=jnp.float32)


def _ab_call(x, w1, w2):
    return pl.pallas_call(
        _ab_body,
        grid=(N // RB,),
        in_specs=[
            pl.BlockSpec((RB, D), lambda i: (i, 0)),
            pl.BlockSpec((D, H), lambda i: (0, 0)),
            pl.BlockSpec((D, H), lambda i: (0, 0)),
        ],
        out_specs=[
            pl.BlockSpec((RB, H), lambda i: (i, 0)),
            pl.BlockSpec((RB, H), lambda i: (i, 0)),
        ],
        out_shape=[
            jax.ShapeDtypeStruct((N, H), jnp.float32),
            jax.ShapeDtypeStruct((NPAD, H), jnp.float32),
        ],
    )(x, w1, w2)


EB = 8000


def _c_body(ea_ref, w3_ref, bp_ref, c_ref):
    c_ref[...] = (jnp.dot(ea_ref[...], w3_ref[...],
                          preferred_element_type=jnp.float32) + bp_ref[...])


def _c_call(ea, w3, bp):
    return pl.pallas_call(
        _c_body,
        grid=(E // EB,),
        in_specs=[
            pl.BlockSpec((EB, DE), lambda i: (i, 0)),
            pl.BlockSpec((DE, H), lambda i: (0, 0)),
            pl.BlockSpec((1, H), lambda i: (0, 0)),
        ],
        out_specs=pl.BlockSpec((EB, H), lambda i: (i, 0)),
        out_shape=jax.ShapeDtypeStruct((E, H), jnp.float32),
    )(ea, w3, bp)


def _stats_body(degb_ref, out_ref):
    col = degb_ref[:, 0:1]
    delta = jnp.sum(jnp.log(col + 1.0)) / N
    dmean = jnp.sum(col) / N
    rows = lax.broadcasted_iota(jnp.int32, (8, 128), 0)
    out_ref[...] = jnp.where(rows < 4, delta, dmean)


def _stats_call(degb):
    return pl.pallas_call(
        _stats_body,
        grid=(1,),
        in_specs=[pl.BlockSpec((N, H), lambda i: (0, 0))],
        out_specs=pl.BlockSpec((8, 128), lambda i: (0, 0)),
        out_shape=jax.ShapeDtypeStruct((8, 128), jnp.float32),
    )(degb)


def _post_body(ssum_ref, ssq_ref, smn_ref, smx_ref, degb_ref, x_ref,
               scal_ref, wp_ref, bp_ref, g_ref, b_ref, o_ref):
    dg = degb_ref[...]
    degc = jnp.maximum(dg, 1.0)
    mean = ssum_ref[...] / degc
    sq = ssq_ref[...] / degc
    std = jnp.sqrt(jnp.maximum(sq - mean * mean, 0.0) + 1e-5)
    pos = dg > 0.0
    mn = jnp.where(pos, smn_ref[...], 0.0)
    mx = jnp.where(pos, smx_ref[...], 0.0)
    delta = scal_ref[0, 0]
    dmean = scal_ref[1, 0]
    amp = jnp.log(dg + 1.0) / (delta + 1e-6)
    lin = dg / (dmean + 1e-6)

    out = jnp.broadcast_to(bp_ref[...], (RB, H))
    for k, t in enumerate((mean, mn, mx, std)):
        out = out + jnp.dot(t, wp_ref[k * H:(k + 1) * H, :],
                            preferred_element_type=jnp.float32)
        out = out + jnp.dot(t * amp, wp_ref[(4 + k) * H:(5 + k) * H, :],
                            preferred_element_type=jnp.float32)
        out = out + jnp.dot(t * lin, wp_ref[(8 + k) * H:(9 + k) * H, :],
                            preferred_element_type=jnp.float32)
    h = x_ref[...] + out
    mu = jnp.mean(h, axis=-1, keepdims=True)
    var = jnp.mean((h - mu) * (h - mu), axis=-1, keepdims=True)
    o_ref[...] = (h - mu) / jnp.sqrt(var + 1e-5) * g_ref[...] + b_ref[...]


def _post_call(ssum, ssq, smn, smx, degb, x, scal, wp, bp, g, b):
    blk = lambda i: (i, 0)
    return pl.pallas_call(
        _post_body,
        grid=(N // RB,),
        in_specs=[
            pl.BlockSpec((RB, H), blk),
            pl.BlockSpec((RB, H), blk),
            pl.BlockSpec((RB, H), blk),
            pl.BlockSpec((RB, H), blk),
            pl.BlockSpec((RB, H), blk),
            pl.BlockSpec((RB, D), blk),
            pl.BlockSpec(memory_space=pltpu.SMEM),
            pl.BlockSpec((12 * H, H), lambda i: (0, 0)),
            pl.BlockSpec((1, H), lambda i: (0, 0)),
            pl.BlockSpec((1, H), lambda i: (0, 0)),
            pl.BlockSpec((1, H), lambda i: (0, 0)),
        ],
        out_specs=pl.BlockSpec((RB, H), blk),
        out_shape=jax.ShapeDtypeStruct((N, H), jnp.float32),
    )(ssum, ssq, smn, smx, degb, x, scal, wp, bp, g, b)


def kernel(x, edge_index, edge_attr, W_pre, b_pre, W_post, b_post, gamma, beta):
    src = edge_index[0]
    dst = edge_index[1]
    w1 = W_pre[:D]
    w2 = W_pre[D:2 * D]
    w3 = W_pre[2 * D:]

    a, b = _ab_call(x, w1, w2)
    c = _c_call(edge_attr, w3, b_pre.reshape(1, H))

    ssum, ssq, smn, smx, deg = _edge_call(src, dst, a, b.reshape(-1), c)
    ssum = ssum.reshape(NPAD, H)
    ssq = ssq.reshape(NPAD, H)
    smn = smn.reshape(NPAD, H)
    smx = smx.reshape(NPAD, H)
    degb = jnp.broadcast_to(deg[:N, None], (N, H))

    stats = _stats_call(degb)
    scal = jnp.stack([stats[0, 0], stats[4, 0]]).reshape(2, 1)

    return _post_call(ssum, ssq, smn, smx, degb, x, scal, W_post,
                      b_post.reshape(1, H), gamma.reshape(1, H),
                      beta.reshape(1, H))


# speedup vs baseline: 1.2821x; 1.0083x over previous
import functools

import jax
import jax.numpy as jnp
from jax import lax
from jax.experimental import pallas as pl
from jax.experimental.pallas import tpu as pltpu
from jax.experimental.pallas import tpu_sc as plsc

N = 10000
E = 320000
D = 128
DE = 16
H = 128

NSLOT = 64
NLOC = 160
NPAD = NSLOT * NLOC
CHUNK = 1280
NCHUNK = E // CHUNK
NV = CHUNK // 16
SUB = 64
CAP = 2048
MASKC = CAP - 1
TRASH = CAP
CBUF = CAP + 16
FBIG = 3.0e38


def _edge_body(src_hbm, dst_hbm, a_hbm, b_hbm, c_hbm,
               ssum_hbm, ssq_hbm, smn_hbm, smx_hbm, deg_hbm,
               dstb, srcb, dstb1, srcb1, idc, srcc, dlocc, bufa, bufc, bloc,
               accs, accq, accmn, accmx, accd, sem_a, sem_c,
               sem_d0, sem_s0, sem_d1, sem_s1):
    wid = lax.axis_index("s") * 2 + lax.axis_index("c")

    zero16 = jnp.zeros((16,), jnp.float32)
    pos16 = jnp.full((16,), FBIG, jnp.float32)
    neg16 = jnp.full((16,), -FBIG, jnp.float32)
    izero16 = jnp.zeros((16,), jnp.int32)

    def _initpad(t, c):
        idc[pl.ds(t * 16, 16)] = izero16
        srcc[pl.ds(t * 16, 16)] = izero16
        dlocc[pl.ds(t * 16, 16)] = izero16
        return c
    lax.fori_loop(0, CBUF // 16, _initpad, 0)

    for p in range(2):
        slot = p * 32 + wid
        lo = slot * NLOC

        def _initacc(t, c):
            o = pl.ds(t * 16, 16)
            accs[o] = zero16
            accq[o] = zero16
            accmn[o] = pos16
            accmx[o] = neg16
            return c
        lax.fori_loop(0, NLOC * H // 16, _initacc, 0)

        def _initd(t, c):
            accd[pl.ds(t * 16, 16)] = zero16
            return c
        lax.fori_loop(0, (NLOC + 16) // 16, _initd, 0)

        pltpu.sync_copy(b_hbm.at[pl.ds(lo * H, NLOC * H)], bloc)

        one16 = jnp.where(lax.iota(jnp.int32, 16) == 0, 1.0, 0.0)

        def _fire(r0):
            pltpu.async_copy(a_hbm.at[srcc.at[pl.ds(r0, SUB)]], bufa, sem_a)
            pltpu.async_copy(c_hbm.at[idc.at[pl.ds(r0, SUB)]], bufc, sem_c)

        def _acc(r0, n):
            pltpu.make_async_copy(
                a_hbm.at[srcc.at[pl.ds(r0, SUB)]], bufa, sem_a).wait()
            pltpu.make_async_copy(
                c_hbm.at[idc.at[pl.ds(r0, SUB)]], bufc, sem_c).wait()

            def _edge(i, c2):
                row = dlocc[pl.ds(r0 + i, 16)][0]
                accd[pl.ds(row, 16)] = accd[pl.ds(row, 16)] + one16
                rb = row * H
                for j in range(H // 16):
                    o = pl.ds(rb + j * 16, 16)
                    a = bufa[i, pl.ds(j * 16, 16)]
                    cc = bufc[i, pl.ds(j * 16, 16)]
                    b = bloc[pl.ds(rb + j * 16, 16)]
                    m = jnp.maximum(a + b + cc, 0.0)
                    accs[o] = accs[o] + m
                    accq[o] = accq[o] + m * m
                    accmn[o] = jnp.minimum(accmn[o], m)
                    accmx[o] = jnp.maximum(accmx[o], m)
                return c2
            lax.fori_loop(0, n, _edge, 0)

        def _window(r0, n):
            _fire(r0)
            _acc(r0, n)

        def _rmask(r):
            return pl.multiple_of(r & MASKC, SUB)

        bufs = ((dstb, srcb, sem_d0, sem_s0),
                (dstb1, srcb1, sem_d1, sem_s1))

        def _start(ci, p):
            db, sb, sd, ss = bufs[p]
            g = ci * CHUNK
            pltpu.async_copy(dst_hbm.at[pl.ds(g, CHUNK)], db, sd)
            pltpu.async_copy(src_hbm.at[pl.ds(g, CHUNK)], sb, ss)

        def _wait(ci, p):
            db, sb, sd, ss = bufs[p]
            g = ci * CHUNK
            pltpu.make_async_copy(dst_hbm.at[pl.ds(g, CHUNK)], db, sd).wait()
            pltpu.make_async_copy(src_hbm.at[pl.ds(g, CHUNK)], sb, ss).wait()

        _start(0, 0)

        def _chunk(ci, wrp):
            w0, r0, pend = wrp
            g = ci * CHUNK

            def _filt_on(db, sb):
                def _filt(v, wv):
                    d = db[pl.ds(v * 16, 16)]
                    s = sb[pl.ds(v * 16, 16)]
                    dl = d - lo
                    msk = (dl >= 0) & (dl < NLOC)
                    eid = lax.iota(jnp.int32, 16) + (g + v * 16)
                    pos = plsc.cumsum(jnp.where(msk, 1, 0))
                    dest = jnp.where(msk, (wv + pos - 1) & MASKC, TRASH)
                    plsc.store_scatter(idc, [dest], eid)
                    plsc.store_scatter(srcc, [dest], s)
                    plsc.store_scatter(dlocc, [dest], dl)
                    return wv + plsc.all_reduce_population_count(msk)
                return lax.fori_loop(0, NV, _filt,
                                     jnp.full((16,), w0, jnp.int32),
                                     unroll=4)

            def _run(p):
                def go():
                    _wait(ci, p)
                    lax.cond(ci + 1 < NCHUNK,
                             lambda: _start(ci + 1, 1 - p), lambda: None)
                    return _filt_on(*bufs[p][:2])
                return go
            wv1 = lax.cond((ci & 1) == 0, _run(0), _run(1))
            w1 = wv1[0]

            r1 = lax.cond(
                pend == 1,
                lambda: (_acc(_rmask(r0), jnp.int32(SUB)), r0 + SUB)[1],
                lambda: r0)

            nwin = (w1 - r1) // SUB

            def _sub(k, r):
                _window(_rmask(r), jnp.int32(SUB))
                return r + SUB
            r2 = lax.fori_loop(0, jnp.maximum(nwin - 1, 0), _sub, r1)
            pend1 = lax.cond(
                nwin >= 1,
                lambda: (_fire(_rmask(r2)), jnp.int32(1))[1],
                lambda: jnp.int32(0))
            return (w1, r2, pend1)
        w, r, pend = lax.fori_loop(
            0, NCHUNK, _chunk,
            (jnp.int32(0), jnp.int32(0), jnp.int32(0)))

        r = lax.cond(
            pend == 1,
            lambda: (_acc(_rmask(r), jnp.int32(SUB)), r + SUB)[1],
            lambda: r)

        def _drain():
            _window(_rmask(r), w - r)
        lax.cond(w > r, _drain, lambda: None)

        pltpu.sync_copy(accs, ssum_hbm.at[pl.ds(lo * H, NLOC * H)])
        pltpu.sync_copy(accq, ssq_hbm.at[pl.ds(lo * H, NLOC * H)])
        pltpu.sync_copy(accmn, smn_hbm.at[pl.ds(lo * H, NLOC * H)])
        pltpu.sync_copy(accmx, smx_hbm.at[pl.ds(lo * H, NLOC * H)])
        pltpu.sync_copy(accd.at[pl.ds(0, NLOC)], deg_hbm.at[pl.ds(lo, NLOC)])


_edge_call = functools.partial(
    pl.kernel,
    out_type=[
        jax.ShapeDtypeStruct((NPAD * H,), jnp.float32),
        jax.ShapeDtypeStruct((NPAD * H,), jnp.float32),
        jax.ShapeDtypeStruct((NPAD * H,), jnp.float32),
        jax.ShapeDtypeStruct((NPAD * H,), jnp.float32),
        jax.ShapeDtypeStruct((NPAD,), jnp.float32),
    ],
    mesh=plsc.VectorSubcoreMesh(core_axis_name="c", subcore_axis_name="s"),
    compiler_params=pltpu.CompilerParams(needs_layout_passes=False),
    scratch_types=[
        pltpu.VMEM((CHUNK,), jnp.int32),
        pltpu.VMEM((CHUNK,), jnp.int32),
        pltpu.VMEM((CHUNK,), jnp.int32),
        pltpu.VMEM((CHUNK,), jnp.int32),
        pltpu.VMEM((CBUF,), jnp.int32),
        pltpu.VMEM((CBUF,), jnp.int32),
        pltpu.VMEM((CBUF,), jnp.int32),
        pltpu.VMEM((SUB, H), jnp.float32),
        pltpu.VMEM((SUB, H), jnp.float32),
        pltpu.VMEM((NLOC * H,), jnp.float32),
        pltpu.VMEM((NLOC * H,), jnp.float32),
        pltpu.VMEM((NLOC * H,), jnp.float32),
        pltpu.VMEM((NLOC * H,), jnp.float32),
        pltpu.VMEM((NLOC * H,), jnp.float32),
        pltpu.VMEM((NLOC + 16,), jnp.float32),
        pltpu.SemaphoreType.DMA,
        pltpu.SemaphoreType.DMA,
        pltpu.SemaphoreType.DMA,
        pltpu.SemaphoreType.DMA,
        pltpu.SemaphoreType.DMA,
        pltpu.SemaphoreType.DMA,
    ],
)(_edge_body)


RB = 400


def _ab_body(x_ref, w1_ref, w2_ref, a_ref, b_ref):
    xb = x_ref[...]
    a_ref[...] = jnp.dot(xb, w1_ref[...], preferred_element_type=jnp.float32)
    b_ref[...] = jnp.dot(xb, w2_ref[...], preferred_element_type=jnp.float32)


def _ab_call(x, w1, w2):
    return pl.pallas_call(
        _ab_body,
        grid=(N // RB,),
        in_specs=[
            pl.BlockSpec((RB, D), lambda i: (i, 0)),
            pl.BlockSpec((D, H), lambda i: (0, 0)),
            pl.BlockSpec((D, H), lambda i: (0, 0)),
        ],
        out_specs=[
            pl.BlockSpec((RB, H), lambda i: (i, 0)),
            pl.BlockSpec((RB, H), lambda i: (i, 0)),
        ],
        out_shape=[
            jax.ShapeDtypeStruct((N, H), jnp.float32),
            jax.ShapeDtypeStruct((NPAD, H), jnp.float32),
        ],
    )(x, w1, w2)


EB = 8000


def _c_body(ea_ref, w3_ref, bp_ref, c_ref):
    c_ref[...] = (jnp.dot(ea_ref[...], w3_ref[...],
                          preferred_element_type=jnp.float32) + bp_ref[...])


def _c_call(ea, w3, bp):
    return pl.pallas_call(
        _c_body,
        grid=(E // EB,),
        in_specs=[
            pl.BlockSpec((EB, DE), lambda i: (i, 0)),
            pl.BlockSpec((DE, H), lambda i: (0, 0)),
            pl.BlockSpec((1, H), lambda i: (0, 0)),
        ],
        out_specs=pl.BlockSpec((EB, H), lambda i: (i, 0)),
        out_shape=jax.ShapeDtypeStruct((E, H), jnp.float32),
    )(ea, w3, bp)


def _stats_body(degb_ref, out_ref):
    col = degb_ref[:, 0:1]
    delta = jnp.sum(jnp.log(col + 1.0)) / N
    dmean = jnp.sum(col) / N
    rows = lax.broadcasted_iota(jnp.int32, (8, 128), 0)
    out_ref[...] = jnp.where(rows < 4, delta, dmean)


def _stats_call(degb):
    return pl.pallas_call(
        _stats_body,
        grid=(1,),
        in_specs=[pl.BlockSpec((N, H), lambda i: (0, 0))],
        out_specs=pl.BlockSpec((8, 128), lambda i: (0, 0)),
        out_shape=jax.ShapeDtypeStruct((8, 128), jnp.float32),
    )(degb)


def _post_body(ssum_ref, ssq_ref, smn_ref, smx_ref, degb_ref, x_ref,
               scal_ref, wp_ref, bp_ref, g_ref, b_ref, o_ref):
    dg = degb_ref[...]
    degc = jnp.maximum(dg, 1.0)
    mean = ssum_ref[...] / degc
    sq = ssq_ref[...] / degc
    std = jnp.sqrt(jnp.maximum(sq - mean * mean, 0.0) + 1e-5)
    pos = dg > 0.0
    mn = jnp.where(pos, smn_ref[...], 0.0)
    mx = jnp.where(pos, smx_ref[...], 0.0)
    delta = scal_ref[0, 0]
    dmean = scal_ref[1, 0]
    amp = jnp.log(dg + 1.0) / (delta + 1e-6)
    lin = dg / (dmean + 1e-6)

    out = jnp.broadcast_to(bp_ref[...], (RB, H))
    for k, t in enumerate((mean, mn, mx, std)):
        out = out + jnp.dot(t, wp_ref[k * H:(k + 1) * H, :],
                            preferred_element_type=jnp.float32)
        out = out + jnp.dot(t * amp, wp_ref[(4 + k) * H:(5 + k) * H, :],
                            preferred_element_type=jnp.float32)
        out = out + jnp.dot(t * lin, wp_ref[(8 + k) * H:(9 + k) * H, :],
                            preferred_element_type=jnp.float32)
    h = x_ref[...] + out
    mu = jnp.mean(h, axis=-1, keepdims=True)
    var = jnp.mean((h - mu) * (h - mu), axis=-1, keepdims=True)
    o_ref[...] = (h - mu) / jnp.sqrt(var + 1e-5) * g_ref[...] + b_ref[...]


def _post_call(ssum, ssq, smn, smx, degb, x, scal, wp, bp, g, b):
    blk = lambda i: (i, 0)
    return pl.pallas_call(
        _post_body,
        grid=(N // RB,),
        in_specs=[
            pl.BlockSpec((RB, H), blk),
            pl.BlockSpec((RB, H), blk),
            pl.BlockSpec((RB, H), blk),
            pl.BlockSpec((RB, H), blk),
            pl.BlockSpec((RB, H), blk),
            pl.BlockSpec((RB, D), blk),
            pl.BlockSpec(memory_space=pltpu.SMEM),
            pl.BlockSpec((12 * H, H), lambda i: (0, 0)),
            pl.BlockSpec((1, H), lambda i: (0, 0)),
            pl.BlockSpec((1, H), lambda i: (0, 0)),
            pl.BlockSpec((1, H), lambda i: (0, 0)),
        ],
        out_specs=pl.BlockSpec((RB, H), blk),
        out_shape=jax.ShapeDtypeStruct((N, H), jnp.float32),
    )(ssum, ssq, smn, smx, degb, x, scal, wp, bp, g, b)


def kernel(x, edge_index, edge_attr, W_pre, b_pre, W_post, b_post, gamma, beta):
    src = edge_index[0]
    dst = edge_index[1]
    w1 = W_pre[:D]
    w2 = W_pre[D:2 * D]
    w3 = W_pre[2 * D:]

    a, b = _ab_call(x, w1, w2)
    c = _c_call(edge_attr, w3, b_pre.reshape(1, H))

    ssum, ssq, smn, smx, deg = _edge_call(src, dst, a, b.reshape(-1), c)
    ssum = ssum.reshape(NPAD, H)
    ssq = ssq.reshape(NPAD, H)
    smn = smn.reshape(NPAD, H)
    smx = smx.reshape(NPAD, H)
    degb = jnp.broadcast_to(deg[:N, None], (N, H))

    stats = _stats_call(degb)
    scal = jnp.stack([stats[0, 0], stats[4, 0]]).reshape(2, 1)

    return _post_call(ssum, ssq, smn, smx, degb, x, scal, W_post,
                      b_post.reshape(1, H), gamma.reshape(1, H),
                      beta.reshape(1, H))


# speedup vs baseline: 1.2939x; 1.0092x over previous
import functools

import jax
import jax.numpy as jnp
from jax import lax
from jax.experimental import pallas as pl
from jax.experimental.pallas import tpu as pltpu
from jax.experimental.pallas import tpu_sc as plsc

N = 10000
E = 320000
D = 128
DE = 16
H = 128

NSLOT = 64
NLOC = 160
NPAD = NSLOT * NLOC
CHUNK = 1280
NCHUNK = E // CHUNK
NV = CHUNK // 16
SUB = 64
CAP = 2048
MASKC = CAP - 1
TRASH = CAP
CBUF = CAP + 16
FBIG = 3.0e38


def _edge_body(src_hbm, dst_hbm, a_hbm, b_hbm, c_hbm,
               ssum_hbm, ssq_hbm, smn_hbm, smx_hbm, deg_hbm,
               dstb, srcb, dstb1, srcb1, idc, srcc, dlocc, bufa, bufc, bloc,
               accs, accq, accmn, accmx, accd, sem_a, sem_c,
               sem_d0, sem_s0, sem_d1, sem_s1):
    wid = lax.axis_index("s") * 2 + lax.axis_index("c")

    zero16 = jnp.zeros((16,), jnp.float32)
    pos16 = jnp.full((16,), FBIG, jnp.float32)
    neg16 = jnp.full((16,), -FBIG, jnp.float32)
    izero16 = jnp.zeros((16,), jnp.int32)

    def _initpad(t, c):
        idc[pl.ds(t * 16, 16)] = izero16
        srcc[pl.ds(t * 16, 16)] = izero16
        dlocc[pl.ds(t * 16, 16)] = izero16
        return c
    lax.fori_loop(0, CBUF // 16, _initpad, 0)

    for p in range(2):
        slot = p * 32 + wid
        lo = slot * NLOC

        def _initacc(t, c):
            o = pl.ds(t * 16, 16)
            accs[o] = zero16
            accq[o] = zero16
            accmn[o] = pos16
            accmx[o] = neg16
            return c
        lax.fori_loop(0, NLOC * H // 16, _initacc, 0)

        def _initd(t, c):
            accd[pl.ds(t * 16, 16)] = zero16
            return c
        lax.fori_loop(0, (NLOC + 16) // 16, _initd, 0)

        pltpu.sync_copy(b_hbm.at[pl.ds(lo * H, NLOC * H)], bloc)

        one16 = jnp.where(lax.iota(jnp.int32, 16) == 0, 1.0, 0.0)

        def _fire(r0):
            pltpu.async_copy(a_hbm.at[srcc.at[pl.ds(r0, SUB)]], bufa, sem_a)
            pltpu.async_copy(c_hbm.at[idc.at[pl.ds(r0, SUB)]], bufc, sem_c)

        def _acc(r0, n):
            pltpu.make_async_copy(
                a_hbm.at[srcc.at[pl.ds(r0, SUB)]], bufa, sem_a).wait()
            pltpu.make_async_copy(
                c_hbm.at[idc.at[pl.ds(r0, SUB)]], bufc, sem_c).wait()

            def _edge(i, c2):
                row = dlocc[pl.ds(r0 + i, 16)][0]
                accd[pl.ds(row, 16)] = accd[pl.ds(row, 16)] + one16
                rb = row * H
                for j in range(H // 16):
                    o = pl.ds(rb + j * 16, 16)
                    a = bufa[i, pl.ds(j * 16, 16)]
                    cc = bufc[i, pl.ds(j * 16, 16)]
                    b = bloc[pl.ds(rb + j * 16, 16)]
                    m = jnp.maximum(a + b + cc, 0.0)
                    accs[o] = accs[o] + m
                    accq[o] = accq[o] + m * m
                    accmn[o] = jnp.minimum(accmn[o], m)
                    accmx[o] = jnp.maximum(accmx[o], m)
                return c2
            lax.fori_loop(0, n, _edge, 0)

        def _window(r0, n):
            _fire(r0)
            _acc(r0, n)

        def _rmask(r):
            return pl.multiple_of(r & MASKC, SUB)

        bufs = ((dstb, srcb, sem_d0, sem_s0),
                (dstb1, srcb1, sem_d1, sem_s1))

        def _start(ci, p):
            db, sb, sd, ss = bufs[p]
            g = ci * CHUNK
            pltpu.async_copy(dst_hbm.at[pl.ds(g, CHUNK)], db, sd)
            pltpu.async_copy(src_hbm.at[pl.ds(g, CHUNK)], sb, ss)

        def _wait(ci, p):
            db, sb, sd, ss = bufs[p]
            g = ci * CHUNK
            pltpu.make_async_copy(dst_hbm.at[pl.ds(g, CHUNK)], db, sd).wait()
            pltpu.make_async_copy(src_hbm.at[pl.ds(g, CHUNK)], sb, ss).wait()

        _start(0, 0)

        def _chunk(ci, wrp):
            w0, r0, pend = wrp
            g = ci * CHUNK

            def _filt_on(db, sb):
                def _filt(v, wv):
                    d = db[pl.ds(v * 16, 16)]
                    s = sb[pl.ds(v * 16, 16)]
                    dl = d - lo
                    msk = (dl >= 0) & (dl < NLOC)
                    eid = lax.iota(jnp.int32, 16) + (g + v * 16)
                    pos = plsc.cumsum(jnp.where(msk, 1, 0))
                    dest = jnp.where(msk, (wv + pos - 1) & MASKC, TRASH)
                    plsc.store_scatter(idc, [dest], eid)
                    plsc.store_scatter(srcc, [dest], s)
                    plsc.store_scatter(dlocc, [dest], dl)
                    return wv + plsc.all_reduce_population_count(msk)
                return lax.fori_loop(0, NV, _filt,
                                     jnp.full((16,), w0, jnp.int32),
                                     unroll=4)

            def _run(p):
                def go():
                    _wait(ci, p)
                    lax.cond(ci + 1 < NCHUNK,
                             lambda: _start(ci + 1, 1 - p), lambda: None)
                    return _filt_on(*bufs[p][:2])
                return go
            wv1 = lax.cond((ci & 1) == 0, _run(0), _run(1))
            w1 = wv1[0]

            navail = (w1 - r0 - pend * SUB) // SUB

            def _consume():
                r1 = lax.cond(
                    pend == 1,
                    lambda: (_acc(_rmask(r0), jnp.int32(SUB)), r0 + SUB)[1],
                    lambda: r0)

                def _sub(k, r):
                    _window(_rmask(r), jnp.int32(SUB))
                    return r + SUB
                r2 = lax.fori_loop(0, navail - 1, _sub, r1)
                _fire(_rmask(r2))
                return (r2, jnp.int32(1))
            r2, pend1 = lax.cond(navail >= 1, _consume,
                                 lambda: (r0, pend))
            return (w1, r2, pend1)
        w, r, pend = lax.fori_loop(
            0, NCHUNK, _chunk,
            (jnp.int32(0), jnp.int32(0), jnp.int32(0)))

        r = lax.cond(
            pend == 1,
            lambda: (_acc(_rmask(r), jnp.int32(SUB)), r + SUB)[1],
            lambda: r)

        def _drain():
            _window(_rmask(r), w - r)
        lax.cond(w > r, _drain, lambda: None)

        pltpu.sync_copy(accs, ssum_hbm.at[pl.ds(lo * H, NLOC * H)])
        pltpu.sync_copy(accq, ssq_hbm.at[pl.ds(lo * H, NLOC * H)])
        pltpu.sync_copy(accmn, smn_hbm.at[pl.ds(lo * H, NLOC * H)])
        pltpu.sync_copy(accmx, smx_hbm.at[pl.ds(lo * H, NLOC * H)])
        pltpu.sync_copy(accd.at[pl.ds(0, NLOC)], deg_hbm.at[pl.ds(lo, NLOC)])


_edge_call = functools.partial(
    pl.kernel,
    out_type=[
        jax.ShapeDtypeStruct((NPAD * H,), jnp.float32),
        jax.ShapeDtypeStruct((NPAD * H,), jnp.float32),
        jax.ShapeDtypeStruct((NPAD * H,), jnp.float32),
        jax.ShapeDtypeStruct((NPAD * H,), jnp.float32),
        jax.ShapeDtypeStruct((NPAD,), jnp.float32),
    ],
    mesh=plsc.VectorSubcoreMesh(core_axis_name="c", subcore_axis_name="s"),
    compiler_params=pltpu.CompilerParams(needs_layout_passes=False),
    scratch_types=[
        pltpu.VMEM((CHUNK,), jnp.int32),
        pltpu.VMEM((CHUNK,), jnp.int32),
        pltpu.VMEM((CHUNK,), jnp.int32),
        pltpu.VMEM((CHUNK,), jnp.int32),
        pltpu.VMEM((CBUF,), jnp.int32),
        pltpu.VMEM((CBUF,), jnp.int32),
        pltpu.VMEM((CBUF,), jnp.int32),
        pltpu.VMEM((SUB, H), jnp.float32),
        pltpu.VMEM((SUB, H), jnp.float32),
        pltpu.VMEM((NLOC * H,), jnp.float32),
        pltpu.VMEM((NLOC * H,), jnp.float32),
        pltpu.VMEM((NLOC * H,), jnp.float32),
        pltpu.VMEM((NLOC * H,), jnp.float32),
        pltpu.VMEM((NLOC * H,), jnp.float32),
        pltpu.VMEM((NLOC + 16,), jnp.float32),
        pltpu.SemaphoreType.DMA,
        pltpu.SemaphoreType.DMA,
        pltpu.SemaphoreType.DMA,
        pltpu.SemaphoreType.DMA,
        pltpu.SemaphoreType.DMA,
        pltpu.SemaphoreType.DMA,
    ],
)(_edge_body)


RB = 400


def _ab_body(x_ref, w1_ref, w2_ref, a_ref, b_ref):
    xb = x_ref[...]
    a_ref[...] = jnp.dot(xb, w1_ref[...], preferred_element_type=jnp.float32)
    b_ref[...] = jnp.dot(xb, w2_ref[...], preferred_element_type=jnp.float32)


def _ab_call(x, w1, w2):
    return pl.pallas_call(
        _ab_body,
        grid=(N // RB,),
        in_specs=[
            pl.BlockSpec((RB, D), lambda i: (i, 0)),
            pl.BlockSpec((D, H), lambda i: (0, 0)),
            pl.BlockSpec((D, H), lambda i: (0, 0)),
        ],
        out_specs=[
            pl.BlockSpec((RB, H), lambda i: (i, 0)),
            pl.BlockSpec((RB, H), lambda i: (i, 0)),
        ],
        out_shape=[
            jax.ShapeDtypeStruct((N, H), jnp.float32),
            jax.ShapeDtypeStruct((NPAD, H), jnp.float32),
        ],
    )(x, w1, w2)


EB = 8000


def _c_body(ea_ref, w3_ref, bp_ref, c_ref):
    c_ref[...] = (jnp.dot(ea_ref[...], w3_ref[...],
                          preferred_element_type=jnp.float32) + bp_ref[...])


def _c_call(ea, w3, bp):
    return pl.pallas_call(
        _c_body,
        grid=(E // EB,),
        in_specs=[
            pl.BlockSpec((EB, DE), lambda i: (i, 0)),
            pl.BlockSpec((DE, H), lambda i: (0, 0)),
            pl.BlockSpec((1, H), lambda i: (0, 0)),
        ],
        out_specs=pl.BlockSpec((EB, H), lambda i: (i, 0)),
        out_shape=jax.ShapeDtypeStruct((E, H), jnp.float32),
    )(ea, w3, bp)


def _stats_body(degb_ref, out_ref):
    col = degb_ref[:, 0:1]
    delta = jnp.sum(jnp.log(col + 1.0)) / N
    dmean = jnp.sum(col) / N
    rows = lax.broadcasted_iota(jnp.int32, (8, 128), 0)
    out_ref[...] = jnp.where(rows < 4, delta, dmean)


def _stats_call(degb):
    return pl.pallas_call(
        _stats_body,
        grid=(1,),
        in_specs=[pl.BlockSpec((N, H), lambda i: (0, 0))],
        out_specs=pl.BlockSpec((8, 128), lambda i: (0, 0)),
        out_shape=jax.ShapeDtypeStruct((8, 128), jnp.float32),
    )(degb)


def _post_body(ssum_ref, ssq_ref, smn_ref, smx_ref, degb_ref, x_ref,
               scal_ref, wp_ref, bp_ref, g_ref, b_ref, o_ref):
    dg = degb_ref[...]
    degc = jnp.maximum(dg, 1.0)
    mean = ssum_ref[...] / degc
    sq = ssq_ref[...] / degc
    std = jnp.sqrt(jnp.maximum(sq - mean * mean, 0.0) + 1e-5)
    pos = dg > 0.0
    mn = jnp.where(pos, smn_ref[...], 0.0)
    mx = jnp.where(pos, smx_ref[...], 0.0)
    delta = scal_ref[0, 0]
    dmean = scal_ref[1, 0]
    amp = jnp.log(dg + 1.0) / (delta + 1e-6)
    lin = dg / (dmean + 1e-6)

    out = jnp.broadcast_to(bp_ref[...], (RB, H))
    for k, t in enumerate((mean, mn, mx, std)):
        out = out + jnp.dot(t, wp_ref[k * H:(k + 1) * H, :],
                            preferred_element_type=jnp.float32)
        out = out + jnp.dot(t * amp, wp_ref[(4 + k) * H:(5 + k) * H, :],
                            preferred_element_type=jnp.float32)
        out = out + jnp.dot(t * lin, wp_ref[(8 + k) * H:(9 + k) * H, :],
                            preferred_element_type=jnp.float32)
    h = x_ref[...] + out
    mu = jnp.mean(h, axis=-1, keepdims=True)
    var = jnp.mean((h - mu) * (h - mu), axis=-1, keepdims=True)
    o_ref[...] = (h - mu) / jnp.sqrt(var + 1e-5) * g_ref[...] + b_ref[...]


def _post_call(ssum, ssq, smn, smx, degb, x, scal, wp, bp, g, b):
    blk = lambda i: (i, 0)
    return pl.pallas_call(
        _post_body,
        grid=(N // RB,),
        in_specs=[
            pl.BlockSpec((RB, H), blk),
            pl.BlockSpec((RB, H), blk),
            pl.BlockSpec((RB, H), blk),
            pl.BlockSpec((RB, H), blk),
            pl.BlockSpec((RB, H), blk),
            pl.BlockSpec((RB, D), blk),
            pl.BlockSpec(memory_space=pltpu.SMEM),
            pl.BlockSpec((12 * H, H), lambda i: (0, 0)),
            pl.BlockSpec((1, H), lambda i: (0, 0)),
            pl.BlockSpec((1, H), lambda i: (0, 0)),
            pl.BlockSpec((1, H), lambda i: (0, 0)),
        ],
        out_specs=pl.BlockSpec((RB, H), blk),
        out_shape=jax.ShapeDtypeStruct((N, H), jnp.float32),
    )(ssum, ssq, smn, smx, degb, x, scal, wp, bp, g, b)


def kernel(x, edge_index, edge_attr, W_pre, b_pre, W_post, b_post, gamma, beta):
    src = edge_index[0]
    dst = edge_index[1]
    w1 = W_pre[:D]
    w2 = W_pre[D:2 * D]
    w3 = W_pre[2 * D:]

    a, b = _ab_call(x, w1, w2)
    c = _c_call(edge_attr, w3, b_pre.reshape(1, H))

    ssum, ssq, smn, smx, deg = _edge_call(src, dst, a, b.reshape(-1), c)
    ssum = ssum.reshape(NPAD, H)
    ssq = ssq.reshape(NPAD, H)
    smn = smn.reshape(NPAD, H)
    smx = smx.reshape(NPAD, H)
    degb = jnp.broadcast_to(deg[:N, None], (N, H))

    stats = _stats_call(degb)
    scal = jnp.stack([stats[0, 0], stats[4, 0]]).reshape(2, 1)

    return _post_call(ssum, ssq, smn, smx, degb, x, scal, W_post,
                      b_post.reshape(1, H), gamma.reshape(1, H),
                      beta.reshape(1, H))
